# baseline (speedup 1.0000x reference)
import functools

import jax
import jax.numpy as jnp
from jax import lax
from jax.experimental import pallas as pl
from jax.experimental.pallas import tpu as pltpu
from jax.experimental.pallas import tpu_sc as plsc

N_NODES = 10000
N_EDGES = 160000
DIM = 256
HALF = 128
N_GROUPS = 64
N_CLASSES = 10

NPAD = 10240
EPAD = 163840
CHUNK = 128
ROWS_A = NPAD // 32
GCHUNK = 80
EDGES_A = EPAD // 32
EDGES_C = EPAD // 16
SLICE_N = NPAD // 16
RB = 2560

def _sc_gather_deg_body(emb_h, xp3_h, dst2_h, ew2_h, h0_h, deg_h,
                        xidx_v, rows0_v, rows1_v, didx2_v, ev2_v, zb_v,
                        deg_s, sem):
    c = lax.axis_index("c")
    s = lax.axis_index("s")
    wid = c * 16 + s
    nchunks = ROWS_A // GCHUNK
    rbufs = (rows0_v, rows1_v)

    for i in range(SLICE_N // 16):
        zb_v[pl.ds(i * 16, 16)] = jnp.zeros((16,), jnp.float32)
    pltpu.sync_copy(zb_v, deg_s.at[pl.ds(s * SLICE_N, SLICE_N)])

    pltpu.sync_copy(dst2_h.at[pl.ds(wid * (EDGES_A // CHUNK),
                                    EDGES_A // CHUNK)], didx2_v)
    pltpu.sync_copy(ew2_h.at[pl.ds(wid * (EDGES_A // CHUNK),
                                   EDGES_A // CHUNK)], ev2_v)

    pltpu.sync_copy(xp3_h.at[wid], xidx_v)
    pltpu.async_copy(emb_h.at[xidx_v.at[0]], rows0_v, sem)
    for k in range(nchunks):
        buf = rbufs[k % 2]
        pltpu.make_async_copy(emb_h.at[xidx_v.at[k]], buf, sem).wait()
        if k + 1 < nchunks:
            pltpu.async_copy(emb_h.at[xidx_v.at[k + 1]],
                             rbufs[(k + 1) % 2], sem)
        base = pl.multiple_of(wid * ROWS_A + k * GCHUNK, 8)
        pltpu.sync_copy(buf, h0_h.at[pl.ds(base, GCHUNK)])

    plsc.subcore_barrier()

    def deg_chunk(k, carry):
        pltpu.async_copy(ev2_v.at[k], deg_s.at[didx2_v.at[k]], sem,
                         add=True)
        return carry
    lax.fori_loop(0, EDGES_A // CHUNK, deg_chunk, 0)

    def deg_drain(k, carry):
        pltpu.make_async_copy(ev2_v.at[k], deg_s.at[didx2_v.at[k]],
                              sem).wait()
        return carry
    lax.fori_loop(0, EDGES_A // CHUNK, deg_drain, 0)

    plsc.subcore_barrier()
    out_base = pl.multiple_of(c * NPAD + s * SLICE_N, 8)
    pltpu.sync_copy(deg_s.at[pl.ds(s * SLICE_N, SLICE_N)],
                    deg_h.at[pl.ds(out_base, SLICE_N)])


GRP = 10
NGRP = EDGES_C // CHUNK // GRP


def _sc_msg_body(tlo_h, thi_h, src4_h, dst4_h, ew4_h, alo_h, ahi_h,
                 sg0, dg0, eg0, sg1, dg1, eg1, rows0_v, rows1_v, acc_s,
                 sem_g, sem_g1, sem_s0, sem_s1, sem_e):
    c = lax.axis_index("c")
    s = lax.axis_index("s")

    def half_body(tbl_h, out_h):
        pltpu.sync_copy(tbl_h.at[pl.ds(s * SLICE_N, SLICE_N)],
                        acc_s.at[pl.ds(s * SLICE_N, SLICE_N)])
        plsc.subcore_barrier()

        def stage(g, sg, dg, eg):
            pltpu.async_copy(src4_h.at[s, g], sg, sem_e)
            pltpu.async_copy(dst4_h.at[s, g], dg, sem_e)
            pltpu.async_copy(ew4_h.at[s, g], eg, sem_e)

        def wait_stage(g, sg, dg, eg):
            pltpu.make_async_copy(src4_h.at[s, g], sg, sem_e).wait()
            pltpu.make_async_copy(dst4_h.at[s, g], dg, sem_e).wait()
            pltpu.make_async_copy(ew4_h.at[s, g], eg, sem_e).wait()

        def start_g(sg, kk, buf, sem):
            pltpu.async_copy(tbl_h.at[sg.at[kk]], buf, sem)

        def wait_g(sg, kk, buf, sem):
            pltpu.make_async_copy(tbl_h.at[sg.at[kk]], buf, sem).wait()

        def scale(eg, kk, buf):
            def gbody(g, carry):
                wvec = eg[kk, pl.ds(g * 16, 16)]
                for l in range(16):
                    w = wvec[l]
                    e = g * 16 + l
                    for j in range(HALF // 16):
                        sl = pl.ds(j * 16, 16)
                        buf[e, sl] = buf[e, sl] * w
                return carry
            lax.fori_loop(0, CHUNK // 16, gbody, 0)

        def start_sc(dg, kk, buf, sem):
            pltpu.async_copy(buf, acc_s.at[dg.at[kk]], sem, add=True)

        def wait_sc(dg, kk, buf, sem):
            pltpu.make_async_copy(buf, acc_s.at[dg.at[kk]], sem).wait()

        def do_group(sg, dg, eg):
            def cpair(q, carry):
                k0 = 2 * q

                @pl.when(q > 0)
                def _():
                    wait_sc(dg, k0 - 1, rows1_v, sem_s1)

                start_g(sg, k0 + 1, rows1_v, sem_g1)
                wait_g(sg, k0, rows0_v, sem_g)
                scale(eg, k0, rows0_v)
                start_sc(dg, k0, rows0_v, sem_s0)
                wait_g(sg, k0 + 1, rows1_v, sem_g1)
                scale(eg, k0 + 1, rows1_v)
                wait_sc(dg, k0, rows0_v, sem_s0)

                @pl.when(q < GRP // 2 - 1)
                def _():
                    start_g(sg, k0 + 2, rows0_v, sem_g)

                start_sc(dg, k0 + 1, rows1_v, sem_s1)
                return carry
            lax.fori_loop(0, GRP // 2, cpair, 0)
            wait_sc(dg, GRP - 1, rows1_v, sem_s1)

        stage(0, sg0, dg0, eg0)
        wait_stage(0, sg0, dg0, eg0)
        start_g(sg0, 0, rows0_v, sem_g)

        def gpair(q, carry):
            g0 = 2 * q
            stage(g0 + 1, sg1, dg1, eg1)
            do_group(sg0, dg0, eg0)
            wait_stage(g0 + 1, sg1, dg1, eg1)
            start_g(sg1, 0, rows0_v, sem_g)

            @pl.when(q < NGRP // 2 - 1)
            def _():
                stage(g0 + 2, sg0, dg0, eg0)

            do_group(sg1, dg1, eg1)

            @pl.when(q < NGRP // 2 - 1)
            def _():
                wait_stage(g0 + 2, sg0, dg0, eg0)
                start_g(sg0, 0, rows0_v, sem_g)

            return carry
        lax.fori_loop(0, NGRP // 2, gpair, 0)

        plsc.subcore_barrier()
        pltpu.sync_copy(acc_s.at[pl.ds(s * SLICE_N, SLICE_N)],
                        out_h.at[pl.ds(s * SLICE_N, SLICE_N)])

    @pl.when(c == 0)
    def _():
        half_body(tlo_h, alo_h)

    @pl.when(c == 1)
    def _():
        half_body(thi_h, ahi_h)


@functools.lru_cache(maxsize=1)
def _sc_kernels():
    mesh = plsc.VectorSubcoreMesh(core_axis_name="c", subcore_axis_name="s",
                                  num_cores=2, num_subcores=16)
    gather_deg = pl.kernel(
        _sc_gather_deg_body,
        out_type=(jax.ShapeDtypeStruct((NPAD, DIM), jnp.float32),
                  jax.ShapeDtypeStruct((2 * NPAD,), jnp.float32)),
        mesh=mesh,
        scratch_types=(
            pltpu.VMEM((ROWS_A // GCHUNK, GCHUNK), jnp.int32),
            pltpu.VMEM((GCHUNK, DIM), jnp.float32),
            pltpu.VMEM((GCHUNK, DIM), jnp.float32),
            pltpu.VMEM((EDGES_A // CHUNK, CHUNK), jnp.int32),
            pltpu.VMEM((EDGES_A // CHUNK, CHUNK), jnp.float32),
            pltpu.VMEM((SLICE_N,), jnp.float32),
            pltpu.VMEM_SHARED((NPAD,), jnp.float32),
            pltpu.SemaphoreType.DMA,
        ),
    )
    msg = pl.kernel(
        _sc_msg_body,
        out_type=(jax.ShapeDtypeStruct((NPAD, HALF), jnp.float32),
                  jax.ShapeDtypeStruct((NPAD, HALF), jnp.float32)),
        mesh=mesh,
        scratch_types=(
            pltpu.VMEM((GRP, CHUNK), jnp.int32),
            pltpu.VMEM((GRP, CHUNK), jnp.int32),
            pltpu.VMEM((GRP, CHUNK), jnp.float32),
            pltpu.VMEM((GRP, CHUNK), jnp.int32),
            pltpu.VMEM((GRP, CHUNK), jnp.int32),
            pltpu.VMEM((GRP, CHUNK), jnp.float32),
            pltpu.VMEM((CHUNK, HALF), jnp.float32),
            pltpu.VMEM((CHUNK, HALF), jnp.float32),
            pltpu.VMEM_SHARED((NPAD, HALF), jnp.float32),
            pltpu.SemaphoreType.DMA,
            pltpu.SemaphoreType.DMA,
            pltpu.SemaphoreType.DMA,
            pltpu.SemaphoreType.DMA,
            pltpu.SemaphoreType.DMA,
        ),
    )
    return gather_deg, msg


def _tc1_body(h0_ref, dega_ref, degb_ref, ab_ref, w1_ref,
              lo_ref, hi_ref, dis_ref):
    deg = dega_ref[...] + degb_ref[...] + 1.0
    dis = jnp.where(deg > 0, lax.rsqrt(deg), 0.0)
    sc = ab_ref[...] * dis * 0.001
    hw = jnp.dot(h0_ref[...] * sc, w1_ref[...],
                 preferred_element_type=jnp.float32)
    lo_ref[...] = hw[:, :HALF]
    hi_ref[...] = hw[:, HALF:]
    dis_ref[...] = dis


def _tc2_body(alo_ref, ahi_ref, dis_ref, b1_ref, w2_ref, olo_ref, ohi_ref):
    dis = dis_ref[...]
    h1 = jnp.concatenate([alo_ref[...], ahi_ref[...]], axis=1) * dis \
        + b1_ref[...]
    h1 = jnp.where(h1 > 0, h1, 0.01 * h1)
    hw = dis * jnp.dot(h1, w2_ref[...], preferred_element_type=jnp.float32)
    olo_ref[...] = hw[:, :HALF]
    ohi_ref[...] = hw[:, HALF:]


def _tc3_body(alo_ref, ahi_ref, dis_ref, b2_ref, bf_ref,
              wo1_ref, bo1_ref, wo2_ref, bo2_ref, out_ref, pool_s, cnt_s):
    i = pl.program_id(0)

    @pl.when(i == 0)
    def _():
        pool_s[...] = jnp.zeros_like(pool_s)
        cnt_s[...] = jnp.zeros_like(cnt_s)

    o2 = jnp.concatenate([alo_ref[...], ahi_ref[...]], axis=1) \
        * dis_ref[...] + b2_ref[...]
    gi = lax.broadcasted_iota(jnp.int32, (N_GROUPS, RB), 0)
    m = jnp.where(gi == bf_ref[...], 1.0, 0.0)
    pool_s[...] = pool_s[...] + jnp.dot(m, o2,
                                        preferred_element_type=jnp.float32)
    cnt_s[...] = cnt_s[...] + jnp.sum(m, axis=1, keepdims=True)

    @pl.when(i == pl.num_programs(0) - 1)
    def _():
        pooled = pool_s[...] / jnp.maximum(cnt_s[:, :1], 1.0)
        o = jnp.dot(pooled, wo1_ref[...],
                    preferred_element_type=jnp.float32) + bo1_ref[...]
        o = jnp.where(o > 0, o, 0.01 * o)
        out_ref[...] = jnp.dot(o, wo2_ref[...],
                               preferred_element_type=jnp.float32) + bo2_ref[...]


def _row_spec(cols):
    return pl.BlockSpec((RB, cols), lambda i: (i, 0))


def _full_spec(rows, cols):
    return pl.BlockSpec((rows, cols), lambda i: (0, 0))


_tc1 = pl.pallas_call(
    _tc1_body,
    grid=(NPAD // RB,),
    in_specs=[_row_spec(DIM), _row_spec(1), _row_spec(1), _row_spec(1),
              _full_spec(DIM, DIM)],
    out_specs=[_row_spec(HALF), _row_spec(HALF), _row_spec(1)],
    out_shape=(jax.ShapeDtypeStruct((NPAD, HALF), jnp.float32),
               jax.ShapeDtypeStruct((NPAD, HALF), jnp.float32),
               jax.ShapeDtypeStruct((NPAD, 1), jnp.float32)),
)

_tc2 = pl.pallas_call(
    _tc2_body,
    grid=(NPAD // RB,),
    in_specs=[_row_spec(HALF), _row_spec(HALF), _row_spec(1),
              _full_spec(1, DIM), _full_spec(DIM, DIM)],
    out_specs=[_row_spec(HALF), _row_spec(HALF)],
    out_shape=(jax.ShapeDtypeStruct((NPAD, HALF), jnp.float32),
               jax.ShapeDtypeStruct((NPAD, HALF), jnp.float32)),
)

_tc3 = pl.pallas_call(
    _tc3_body,
    grid=(NPAD // RB,),
    in_specs=[_row_spec(HALF), _row_spec(HALF), _row_spec(1),
              _full_spec(1, DIM), pl.BlockSpec((1, RB), lambda i: (0, i)),
              _full_spec(DIM, N_GROUPS), _full_spec(1, N_GROUPS),
              _full_spec(N_GROUPS, N_CLASSES), _full_spec(1, N_CLASSES)],
    out_specs=pl.BlockSpec((N_GROUPS, N_CLASSES), lambda i: (0, 0)),
    out_shape=jax.ShapeDtypeStruct((N_GROUPS, N_CLASSES), jnp.float32),
    scratch_shapes=[pltpu.VMEM((N_GROUPS, DIM), jnp.float32),
                    pltpu.VMEM((N_GROUPS, HALF), jnp.float32)],
)


def kernel(x, edge_index, edge_attr, abundancies, batch, emb,
           W1, b1, W2, b2, Wo1, bo1, Wo2, bo2):
    f32 = jnp.float32
    i32 = jnp.int32
    xp = jnp.pad(x.astype(i32), (0, NPAD - N_NODES))
    srcp = jnp.pad(edge_index[0].astype(i32), (0, EPAD - N_EDGES))
    dstp = jnp.pad(edge_index[1].astype(i32), (0, EPAD - N_EDGES))
    ewp = jnp.pad(edge_attr.astype(f32), (0, EPAD - N_EDGES))
    abp = jnp.pad(abundancies.astype(f32), (0, NPAD - N_NODES))
    abp = abp.reshape(NPAD, 1)
    bfp = jnp.pad(batch.astype(i32), (0, NPAD - N_NODES),
                  constant_values=N_GROUPS).reshape(1, NPAD)

    xp3 = xp.reshape(32, ROWS_A // GCHUNK, GCHUNK)
    dst2 = dstp.reshape(EPAD // CHUNK, CHUNK)
    ew2 = ewp.reshape(EPAD // CHUNK, CHUNK)
    src4 = srcp.reshape(16, NGRP, GRP, CHUNK)
    dst4 = dstp.reshape(16, NGRP, GRP, CHUNK)
    ew4 = ewp.reshape(16, NGRP, GRP, CHUNK)

    _sc_gather_deg, _sc_msg = _sc_kernels()
    h0, degflat = _sc_gather_deg(emb.astype(f32), xp3, dst2, ew2)
    dega = degflat[:NPAD].reshape(NPAD, 1)
    degb = degflat[NPAD:].reshape(NPAD, 1)

    lo1, hi1, dis = _tc1(h0, dega, degb, abp, W1.astype(f32))
    a1lo, a1hi = _sc_msg(lo1, hi1, src4, dst4, ew4)
    lo2, hi2 = _tc2(a1lo, a1hi, dis, b1.astype(f32).reshape(1, DIM),
                    W2.astype(f32))
    a2lo, a2hi = _sc_msg(lo2, hi2, src4, dst4, ew4)
    out = _tc3(a2lo, a2hi, dis, b2.astype(f32).reshape(1, DIM), bfp,
               Wo1.astype(f32), bo1.astype(f32).reshape(1, N_GROUPS),
               Wo2.astype(f32), bo2.astype(f32).reshape(1, N_CLASSES))
    return out


# speedup vs baseline: 1.0349x; 1.0349x over previous
import functools

import jax
import jax.numpy as jnp
from jax import lax
from jax.experimental import pallas as pl
from jax.experimental.pallas import tpu as pltpu
from jax.experimental.pallas import tpu_sc as plsc

N_NODES = 10000
N_EDGES = 160000
DIM = 256
HALF = 128
N_GROUPS = 64
N_CLASSES = 10

NPAD = 10240
EPAD = 163840
CHUNK = 128
ROWS_A = NPAD // 32
GCHUNK = 80
EDGES_A = EPAD // 32
EDGES_C = EPAD // 16
SLICE_N = NPAD // 16
RB = 2560

def _sc_gather_deg_body(emb_h, xp3_h, dst2_h, ew2_h, h0_h, deg_h,
                        xidx_v, rows0_v, rows1_v, didx2_v, ev2_v, zb_v,
                        deg_s, sem):
    c = lax.axis_index("c")
    s = lax.axis_index("s")
    wid = c * 16 + s
    nchunks = ROWS_A // GCHUNK
    rbufs = (rows0_v, rows1_v)

    for i in range(SLICE_N // 16):
        zb_v[pl.ds(i * 16, 16)] = jnp.zeros((16,), jnp.float32)
    pltpu.sync_copy(zb_v, deg_s.at[pl.ds(s * SLICE_N, SLICE_N)])

    pltpu.sync_copy(dst2_h.at[pl.ds(wid * (EDGES_A // CHUNK),
                                    EDGES_A // CHUNK)], didx2_v)
    pltpu.sync_copy(ew2_h.at[pl.ds(wid * (EDGES_A // CHUNK),
                                   EDGES_A // CHUNK)], ev2_v)

    pltpu.sync_copy(xp3_h.at[wid], xidx_v)
    pltpu.async_copy(emb_h.at[xidx_v.at[0]], rows0_v, sem)
    for k in range(nchunks):
        buf = rbufs[k % 2]
        pltpu.make_async_copy(emb_h.at[xidx_v.at[k]], buf, sem).wait()
        if k + 1 < nchunks:
            pltpu.async_copy(emb_h.at[xidx_v.at[k + 1]],
                             rbufs[(k + 1) % 2], sem)
        base = pl.multiple_of(wid * ROWS_A + k * GCHUNK, 8)
        pltpu.sync_copy(buf, h0_h.at[pl.ds(base, GCHUNK)])

    plsc.subcore_barrier()

    def deg_chunk(k, carry):
        pltpu.async_copy(ev2_v.at[k], deg_s.at[didx2_v.at[k]], sem,
                         add=True)
        return carry
    lax.fori_loop(0, EDGES_A // CHUNK, deg_chunk, 0)

    def deg_drain(k, carry):
        pltpu.make_async_copy(ev2_v.at[k], deg_s.at[didx2_v.at[k]],
                              sem).wait()
        return carry
    lax.fori_loop(0, EDGES_A // CHUNK, deg_drain, 0)

    plsc.subcore_barrier()
    out_base = pl.multiple_of(c * NPAD + s * SLICE_N, 8)
    pltpu.sync_copy(deg_s.at[pl.ds(s * SLICE_N, SLICE_N)],
                    deg_h.at[pl.ds(out_base, SLICE_N)])


GRP = 10
NGRP = EDGES_C // CHUNK // GRP


def _sc_msg_body(tlo_h, thi_h, src4_h, dst4_h, ew4_h, alo_h, ahi_h,
                 sg0, dg0, eg0, sg1, dg1, eg1, rows0_v, rows1_v, acc_s,
                 sem_g, sem_g1, sem_e):
    c = lax.axis_index("c")
    s = lax.axis_index("s")

    def half_body(tbl_h, out_h):
        pltpu.sync_copy(tbl_h.at[pl.ds(s * SLICE_N, SLICE_N)],
                        acc_s.at[pl.ds(s * SLICE_N, SLICE_N)])
        plsc.subcore_barrier()

        def stage(g, sg, dg, eg):
            pltpu.async_copy(src4_h.at[s, g], sg, sem_e)
            pltpu.async_copy(dst4_h.at[s, g], dg, sem_e)
            pltpu.async_copy(ew4_h.at[s, g], eg, sem_e)

        def wait_stage(g, sg, dg, eg):
            pltpu.make_async_copy(src4_h.at[s, g], sg, sem_e).wait()
            pltpu.make_async_copy(dst4_h.at[s, g], dg, sem_e).wait()
            pltpu.make_async_copy(ew4_h.at[s, g], eg, sem_e).wait()

        def start_g(sg, kk, buf, sem):
            pltpu.async_copy(tbl_h.at[sg.at[kk]], buf, sem)

        def wait_g(sg, kk, buf, sem):
            pltpu.make_async_copy(tbl_h.at[sg.at[kk]], buf, sem).wait()

        def scale(eg, kk, buf):
            def gbody(g, carry):
                wvec = eg[kk, pl.ds(g * 16, 16)]
                for l in range(16):
                    w = wvec[l]
                    e = g * 16 + l
                    for j in range(HALF // 16):
                        sl = pl.ds(j * 16, 16)
                        buf[e, sl] = buf[e, sl] * w
                return carry
            lax.fori_loop(0, CHUNK // 16, gbody, 0)

        def scatter(dg, kk, buf):
            pltpu.sync_copy(buf, acc_s.at[dg.at[kk]], add=True)

        def do_group(sg, dg, eg):
            def cpair(q, carry):
                k0 = 2 * q
                start_g(sg, k0 + 1, rows1_v, sem_g1)
                wait_g(sg, k0, rows0_v, sem_g)
                scale(eg, k0, rows0_v)
                scatter(dg, k0, rows0_v)

                @pl.when(q < GRP // 2 - 1)
                def _():
                    start_g(sg, k0 + 2, rows0_v, sem_g)

                wait_g(sg, k0 + 1, rows1_v, sem_g1)
                scale(eg, k0 + 1, rows1_v)
                scatter(dg, k0 + 1, rows1_v)
                return carry
            lax.fori_loop(0, GRP // 2, cpair, 0)

        stage(0, sg0, dg0, eg0)
        wait_stage(0, sg0, dg0, eg0)
        start_g(sg0, 0, rows0_v, sem_g)

        def gpair(q, carry):
            g0 = 2 * q
            stage(g0 + 1, sg1, dg1, eg1)
            do_group(sg0, dg0, eg0)
            wait_stage(g0 + 1, sg1, dg1, eg1)
            start_g(sg1, 0, rows0_v, sem_g)

            @pl.when(q < NGRP // 2 - 1)
            def _():
                stage(g0 + 2, sg0, dg0, eg0)

            do_group(sg1, dg1, eg1)

            @pl.when(q < NGRP // 2 - 1)
            def _():
                wait_stage(g0 + 2, sg0, dg0, eg0)
                start_g(sg0, 0, rows0_v, sem_g)

            return carry
        lax.fori_loop(0, NGRP // 2, gpair, 0)

        plsc.subcore_barrier()
        pltpu.sync_copy(acc_s.at[pl.ds(s * SLICE_N, SLICE_N)],
                        out_h.at[pl.ds(s * SLICE_N, SLICE_N)])

    @pl.when(c == 0)
    def _():
        half_body(tlo_h, alo_h)

    @pl.when(c == 1)
    def _():
        half_body(thi_h, ahi_h)


@functools.lru_cache(maxsize=1)
def _sc_kernels():
    mesh = plsc.VectorSubcoreMesh(core_axis_name="c", subcore_axis_name="s",
                                  num_cores=2, num_subcores=16)
    gather_deg = pl.kernel(
        _sc_gather_deg_body,
        out_type=(jax.ShapeDtypeStruct((NPAD, DIM), jnp.float32),
                  jax.ShapeDtypeStruct((2 * NPAD,), jnp.float32)),
        mesh=mesh,
        scratch_types=(
            pltpu.VMEM((ROWS_A // GCHUNK, GCHUNK), jnp.int32),
            pltpu.VMEM((GCHUNK, DIM), jnp.float32),
            pltpu.VMEM((GCHUNK, DIM), jnp.float32),
            pltpu.VMEM((EDGES_A // CHUNK, CHUNK), jnp.int32),
            pltpu.VMEM((EDGES_A // CHUNK, CHUNK), jnp.float32),
            pltpu.VMEM((SLICE_N,), jnp.float32),
            pltpu.VMEM_SHARED((NPAD,), jnp.float32),
            pltpu.SemaphoreType.DMA,
        ),
    )
    msg = pl.kernel(
        _sc_msg_body,
        out_type=(jax.ShapeDtypeStruct((NPAD, HALF), jnp.float32),
                  jax.ShapeDtypeStruct((NPAD, HALF), jnp.float32)),
        mesh=mesh,
        scratch_types=(
            pltpu.VMEM((GRP, CHUNK), jnp.int32),
            pltpu.VMEM((GRP, CHUNK), jnp.int32),
            pltpu.VMEM((GRP, CHUNK), jnp.float32),
            pltpu.VMEM((GRP, CHUNK), jnp.int32),
            pltpu.VMEM((GRP, CHUNK), jnp.int32),
            pltpu.VMEM((GRP, CHUNK), jnp.float32),
            pltpu.VMEM((CHUNK, HALF), jnp.float32),
            pltpu.VMEM((CHUNK, HALF), jnp.float32),
            pltpu.VMEM_SHARED((NPAD, HALF), jnp.float32),
            pltpu.SemaphoreType.DMA,
            pltpu.SemaphoreType.DMA,
            pltpu.SemaphoreType.DMA,
        ),
    )
    return gather_deg, msg


def _tc1_body(h0_ref, dega_ref, degb_ref, ab_ref, w1_ref,
              lo_ref, hi_ref, dis_ref):
    deg = dega_ref[...] + degb_ref[...] + 1.0
    dis = jnp.where(deg > 0, lax.rsqrt(deg), 0.0)
    sc = ab_ref[...] * dis * 0.001
    hw = jnp.dot(h0_ref[...] * sc, w1_ref[...],
                 preferred_element_type=jnp.float32)
    lo_ref[...] = hw[:, :HALF]
    hi_ref[...] = hw[:, HALF:]
    dis_ref[...] = dis


def _tc2_body(alo_ref, ahi_ref, dis_ref, b1_ref, w2_ref, olo_ref, ohi_ref):
    dis = dis_ref[...]
    h1 = jnp.concatenate([alo_ref[...], ahi_ref[...]], axis=1) * dis \
        + b1_ref[...]
    h1 = jnp.where(h1 > 0, h1, 0.01 * h1)
    hw = dis * jnp.dot(h1, w2_ref[...], preferred_element_type=jnp.float32)
    olo_ref[...] = hw[:, :HALF]
    ohi_ref[...] = hw[:, HALF:]


def _tc3_body(alo_ref, ahi_ref, dis_ref, b2_ref, bf_ref,
              wo1_ref, bo1_ref, wo2_ref, bo2_ref, out_ref, pool_s, cnt_s):
    i = pl.program_id(0)

    @pl.when(i == 0)
    def _():
        pool_s[...] = jnp.zeros_like(pool_s)
        cnt_s[...] = jnp.zeros_like(cnt_s)

    o2 = jnp.concatenate([alo_ref[...], ahi_ref[...]], axis=1) \
        * dis_ref[...] + b2_ref[...]
    gi = lax.broadcasted_iota(jnp.int32, (N_GROUPS, RB), 0)
    m = jnp.where(gi == bf_ref[...], 1.0, 0.0)
    pool_s[...] = pool_s[...] + jnp.dot(m, o2,
                                        preferred_element_type=jnp.float32)
    cnt_s[...] = cnt_s[...] + jnp.sum(m, axis=1, keepdims=True)

    @pl.when(i == pl.num_programs(0) - 1)
    def _():
        pooled = pool_s[...] / jnp.maximum(cnt_s[:, :1], 1.0)
        o = jnp.dot(pooled, wo1_ref[...],
                    preferred_element_type=jnp.float32) + bo1_ref[...]
        o = jnp.where(o > 0, o, 0.01 * o)
        out_ref[...] = jnp.dot(o, wo2_ref[...],
                               preferred_element_type=jnp.float32) + bo2_ref[...]


def _row_spec(cols):
    return pl.BlockSpec((RB, cols), lambda i: (i, 0))


def _full_spec(rows, cols):
    return pl.BlockSpec((rows, cols), lambda i: (0, 0))


_tc1 = pl.pallas_call(
    _tc1_body,
    grid=(NPAD // RB,),
    in_specs=[_row_spec(DIM), _row_spec(1), _row_spec(1), _row_spec(1),
              _full_spec(DIM, DIM)],
    out_specs=[_row_spec(HALF), _row_spec(HALF), _row_spec(1)],
    out_shape=(jax.ShapeDtypeStruct((NPAD, HALF), jnp.float32),
               jax.ShapeDtypeStruct((NPAD, HALF), jnp.float32),
               jax.ShapeDtypeStruct((NPAD, 1), jnp.float32)),
)

_tc2 = pl.pallas_call(
    _tc2_body,
    grid=(NPAD // RB,),
    in_specs=[_row_spec(HALF), _row_spec(HALF), _row_spec(1),
              _full_spec(1, DIM), _full_spec(DIM, DIM)],
    out_specs=[_row_spec(HALF), _row_spec(HALF)],
    out_shape=(jax.ShapeDtypeStruct((NPAD, HALF), jnp.float32),
               jax.ShapeDtypeStruct((NPAD, HALF), jnp.float32)),
)

_tc3 = pl.pallas_call(
    _tc3_body,
    grid=(NPAD // RB,),
    in_specs=[_row_spec(HALF), _row_spec(HALF), _row_spec(1),
              _full_spec(1, DIM), pl.BlockSpec((1, RB), lambda i: (0, i)),
              _full_spec(DIM, N_GROUPS), _full_spec(1, N_GROUPS),
              _full_spec(N_GROUPS, N_CLASSES), _full_spec(1, N_CLASSES)],
    out_specs=pl.BlockSpec((N_GROUPS, N_CLASSES), lambda i: (0, 0)),
    out_shape=jax.ShapeDtypeStruct((N_GROUPS, N_CLASSES), jnp.float32),
    scratch_shapes=[pltpu.VMEM((N_GROUPS, DIM), jnp.float32),
                    pltpu.VMEM((N_GROUPS, HALF), jnp.float32)],
)


def kernel(x, edge_index, edge_attr, abundancies, batch, emb,
           W1, b1, W2, b2, Wo1, bo1, Wo2, bo2):
    f32 = jnp.float32
    i32 = jnp.int32
    xp = jnp.pad(x.astype(i32), (0, NPAD - N_NODES))
    srcp = jnp.pad(edge_index[0].astype(i32), (0, EPAD - N_EDGES))
    dstp = jnp.pad(edge_index[1].astype(i32), (0, EPAD - N_EDGES))
    ewp = jnp.pad(edge_attr.astype(f32), (0, EPAD - N_EDGES))
    abp = jnp.pad(abundancies.astype(f32), (0, NPAD - N_NODES))
    abp = abp.reshape(NPAD, 1)
    bfp = jnp.pad(batch.astype(i32), (0, NPAD - N_NODES),
                  constant_values=N_GROUPS).reshape(1, NPAD)

    xp3 = xp.reshape(32, ROWS_A // GCHUNK, GCHUNK)
    dst2 = dstp.reshape(EPAD // CHUNK, CHUNK)
    ew2 = ewp.reshape(EPAD // CHUNK, CHUNK)
    src4 = srcp.reshape(16, NGRP, GRP, CHUNK)
    dst4 = dstp.reshape(16, NGRP, GRP, CHUNK)
    ew4 = ewp.reshape(16, NGRP, GRP, CHUNK)

    _sc_gather_deg, _sc_msg = _sc_kernels()
    h0, degflat = _sc_gather_deg(emb.astype(f32), xp3, dst2, ew2)
    dega = degflat[:NPAD].reshape(NPAD, 1)
    degb = degflat[NPAD:].reshape(NPAD, 1)

    lo1, hi1, dis = _tc1(h0, dega, degb, abp, W1.astype(f32))
    a1lo, a1hi = _sc_msg(lo1, hi1, src4, dst4, ew4)
    lo2, hi2 = _tc2(a1lo, a1hi, dis, b1.astype(f32).reshape(1, DIM),
                    W2.astype(f32))
    a2lo, a2hi = _sc_msg(lo2, hi2, src4, dst4, ew4)
    out = _tc3(a2lo, a2hi, dis, b2.astype(f32).reshape(1, DIM), bfp,
               Wo1.astype(f32), bo1.astype(f32).reshape(1, N_GROUPS),
               Wo2.astype(f32), bo2.astype(f32).reshape(1, N_CLASSES))
    return out
